# Initial kernel scaffold; baseline (speedup 1.0000x reference)
import functools

import jax
import jax.numpy as jnp
from jax import lax
from jax.experimental import pallas as pl
from jax.experimental.pallas import tpu as pltpu
from jax.experimental.pallas import tpu_sc as plsc

_B = 16384
_EMB = 50
_HID = 64

_info = plsc.get_sparse_core_info()
_NW = _info.num_cores * _info.num_subcores
_BPW = _B // _NW

_mesh = plsc.VectorSubcoreMesh(core_axis_name="c", subcore_axis_name="s")


@functools.partial(
    pl.kernel,
    out_type=[jax.ShapeDtypeStruct((_B, _EMB), jnp.float32)] * 4,
    mesh=_mesh,
    scratch_types=[
        pltpu.VMEM((_BPW,), jnp.int32),
        pltpu.VMEM((_BPW, _EMB), jnp.float32),
        pltpu.SemaphoreType.DMA,
    ],
)
def _gather4(uid, ugen, mid, mgen, ut, mt, gt,
             out_u, out_ug, out_m, out_mg, idx_v, rows_v, sem):
    wid = lax.axis_index("s") * _info.num_cores + lax.axis_index("c")
    base = wid * _BPW
    for idx_hbm, table, out in ((uid, ut, out_u), (ugen, gt, out_ug),
                                (mid, mt, out_m), (mgen, gt, out_mg)):
        pltpu.sync_copy(idx_hbm.at[pl.ds(base, _BPW)], idx_v)
        pltpu.async_copy(table.at[idx_v], rows_v, sem).wait()
        pltpu.sync_copy(rows_v, out.at[pl.ds(base, _BPW)])


_BLK = 2048


def _mlp_body(u, ug, m, mg, w1a, w1b, w1c, w1d, b1, w2t, b2, out):
    acc = jnp.dot(u[...], w1a[...], preferred_element_type=jnp.float32)
    acc += jnp.dot(ug[...], w1b[...], preferred_element_type=jnp.float32)
    acc += jnp.dot(m[...], w1c[...], preferred_element_type=jnp.float32)
    acc += jnp.dot(mg[...], w1d[...], preferred_element_type=jnp.float32)
    h = jnp.maximum(acc + b1[...], 0.0)
    o = jnp.sum(h * w2t[...], axis=1, keepdims=True) + b2[...]
    out[...] = jax.nn.sigmoid(o)


def _mlp(u, ug, m, mg, W1, b1, W2, b2):
    w1a, w1b, w1c, w1d = (W1[0:50], W1[50:100], W1[100:150], W1[150:200])
    b1r = b1.reshape(1, _HID)
    w2t = W2.reshape(1, _HID)
    b2r = b2.reshape(1, 1)
    wspec = pl.BlockSpec((_EMB, _HID), lambda i: (0, 0))
    hspec = pl.BlockSpec((1, _HID), lambda i: (0, 0))
    xspec = pl.BlockSpec((_BLK, _EMB), lambda i: (i, 0))
    return pl.pallas_call(
        _mlp_body,
        grid=(_B // _BLK,),
        in_specs=[xspec, xspec, xspec, xspec,
                  wspec, wspec, wspec, wspec,
                  hspec, hspec, pl.BlockSpec((1, 1), lambda i: (0, 0))],
        out_specs=pl.BlockSpec((_BLK, 1), lambda i: (i, 0)),
        out_shape=jax.ShapeDtypeStruct((_B, 1), jnp.float32),
    )(u, ug, m, mg, w1a, w1b, w1c, w1d, b1r, w2t, b2r)


def kernel(user_id, user_age, user_genre, movie_id, movie_rating, movie_genre,
           movie_year, user_table, movie_table, genre_table, W1, b1, W2, b2):
    uid = user_id[:, 0]
    ugen = user_genre[:, 0]
    mid = movie_id[:, 0]
    mgen = movie_genre[:, 0]
    u, ug, m, mg = _gather4(uid, ugen, mid, mgen,
                            user_table, movie_table, genre_table)
    out = _mlp(u, ug, m, mg, W1, b1, W2, b2)
    return out.reshape(_B, 1, 1)


# speedup vs baseline: 2.6675x; 2.6675x over previous
import jax
import jax.numpy as jnp
from jax.experimental import pallas as pl

_B = 16384
_D = 200
_HID = 64
_BLKN = 2048


def _mlp_body(x, w1t, b1, w2t, b2, out):
    h = jnp.maximum(
        jnp.dot(w1t[...], x[...], preferred_element_type=jnp.float32)
        + b1[...], 0.0)
    o = jnp.dot(w2t[...], h, preferred_element_type=jnp.float32) + b2[...]
    out[...] = jax.nn.sigmoid(o)


def _mlp(xT, W1, b1, W2, b2):
    w1t = W1.T
    b1c = b1.reshape(_HID, 1)
    w2t = W2.reshape(1, _HID)
    b2r = b2.reshape(1, 1)
    return pl.pallas_call(
        _mlp_body,
        grid=(_B // _BLKN,),
        in_specs=[pl.BlockSpec((_D, _BLKN), lambda i: (0, i)),
                  pl.BlockSpec((_HID, _D), lambda i: (0, 0)),
                  pl.BlockSpec((_HID, 1), lambda i: (0, 0)),
                  pl.BlockSpec((1, _HID), lambda i: (0, 0)),
                  pl.BlockSpec((1, 1), lambda i: (0, 0))],
        out_specs=pl.BlockSpec((1, _BLKN), lambda i: (0, i)),
        out_shape=jax.ShapeDtypeStruct((1, _B), jnp.float32),
    )(xT, w1t, b1c, w2t, b2r)


def kernel(user_id, user_age, user_genre, movie_id, movie_rating, movie_genre,
           movie_year, user_table, movie_table, genre_table, W1, b1, W2, b2):
    u = jnp.take(user_table, user_id[:, 0], axis=0)
    ug = jnp.take(genre_table, user_genre[:, 0], axis=0)
    m = jnp.take(movie_table, movie_id[:, 0], axis=0)
    mg = jnp.take(genre_table, movie_genre[:, 0], axis=0)
    xT = jnp.concatenate([u, ug, m, mg], axis=-1).T
    out = _mlp(xT, W1, b1, W2, b2)
    return out.reshape(_B, 1, 1)
